# baseline (speedup 1.0000x reference)
import functools

import jax
import jax.numpy as jnp
from jax import lax
from jax.experimental import pallas as pl
from jax.experimental.pallas import tpu as pltpu
from jax.experimental.pallas import tpu_sc as plsc

DIM = 2048
INTER = 1408
NEXP = 8
TOPK = 2
SHARED_INTER = 2 * INTER
T = 2048
NASN = T * TOPK
BROW = 128
PAD_N = NASN + NEXP * BROW
NBLK = PAD_N // BROW

GATE_BT = 512
SH_BT = 256
SH_IB = 256
SH_NI = SHARED_INTER // SH_IB


SC_NT = 16
SC_CHUNK = NASN // SC_NT
PAD_SLICE = PAD_N // SC_NT
NBLK_PAD = 48

_DISPATCH_MESH = plsc.VectorSubcoreMesh(
    core_axis_name="c", subcore_axis_name="s", num_cores=1)


def _dispatch_body(eflat_hbm, pos_hbm, stok_hbm, bexp_hbm,
                   e_v, pos_v, tok_v, run_v, ends_v, zero_v, hist_me,
                   hist_all, bev_v, hist_sh, sem):
    wid = lax.axis_index("s")
    lanes = lax.iota(jnp.int32, 16)
    pltpu.sync_copy(eflat_hbm.at[pl.ds(wid * SC_CHUNK, SC_CHUNK)], e_v)

    hist = jnp.zeros((16,), jnp.int32)
    for e in range(NEXP):
        cnt = jnp.zeros((16,), jnp.int32)
        for j in range(SC_CHUNK // 16):
            ev = e_v[pl.ds(j * 16, 16)]
            cnt = cnt + plsc.all_reduce_population_count(ev == e)
        hist = jnp.where(lanes == e, cnt, hist)
    hist_me[...] = hist

    pltpu.sync_copy(hist_me, hist_sh.at[pl.ds(wid * 16, 16)])
    plsc.subcore_barrier()
    pltpu.sync_copy(hist_sh, hist_all)
    counts = jnp.zeros((16,), jnp.int32)
    prefix = jnp.zeros((16,), jnp.int32)
    widv = jnp.full((16,), wid, jnp.int32)
    for t in range(SC_NT):
        row = hist_all[pl.ds(t * 16, 16)]
        counts = counts + row
        prefix = prefix + jnp.where(jnp.full((16,), t, jnp.int32) < widv,
                                    row, 0)
    padded = ((counts + (BROW - 1)) >> 7) << 7
    ends = plsc.cumsum(padded)
    ends_v[...] = ends
    run_v[...] = (ends - padded) + prefix

    ibase = wid * SC_CHUNK
    for j in range(SC_CHUNK // 16):
        ev = e_v[pl.ds(j * 16, 16)]
        blane = plsc.load_gather(run_v, [ev])
        rank = jnp.zeros((16,), jnp.int32)
        newcnt = jnp.zeros((16,), jnp.int32)
        for e in range(NEXP):
            m = ev == e
            cs = plsc.cumsum(m.astype(jnp.int32))
            rank = rank + jnp.where(m, cs - 1, 0)
            newcnt = newcnt + jnp.where(
                lanes == e, plsc.all_reduce_population_count(m), 0)
        pos_v[pl.ds(j * 16, 16)] = blane + rank
        tok_v[pl.ds(j * 16, 16)] = (ibase + j * 16 + lanes) // TOPK
        run_v[...] = run_v[...] + newcnt
    pltpu.sync_copy(pos_v, pos_hbm.at[pl.ds(ibase, SC_CHUNK)])

    for k in range(PAD_SLICE // 16):
        zero_v[pl.ds(k * 16, 16)] = jnp.zeros((16,), jnp.int32)
    pltpu.sync_copy(zero_v, stok_hbm.at[pl.ds(wid * PAD_SLICE, PAD_SLICE)])
    plsc.subcore_barrier()
    pltpu.async_copy(tok_v, stok_hbm.at[pos_v], sem).wait()

    @pl.when(wid == 0)
    def _():
        endsl = ends_v[...]
        for v in range(NBLK_PAD // 16):
            start = (v * 16 + lanes) * BROW
            be = jnp.zeros((16,), jnp.int32)
            for e in range(NEXP):
                be = be + (start >= jnp.full((16,), endsl[e])).astype(
                    jnp.int32)
            bev_v[pl.ds(v * 16, 16)] = jnp.minimum(be, NEXP - 1)
        pltpu.sync_copy(bev_v, bexp_hbm)


@functools.partial(
    pl.kernel,
    out_type=[
        jax.ShapeDtypeStruct((NASN,), jnp.int32),
        jax.ShapeDtypeStruct((PAD_N,), jnp.int32),
        jax.ShapeDtypeStruct((NBLK_PAD,), jnp.int32),
    ],
    mesh=_DISPATCH_MESH,
    compiler_params=pltpu.CompilerParams(needs_layout_passes=False),
    scratch_types=[
        pltpu.VMEM((SC_CHUNK,), jnp.int32),
        pltpu.VMEM((SC_CHUNK,), jnp.int32),
        pltpu.VMEM((SC_CHUNK,), jnp.int32),
        pltpu.VMEM((16,), jnp.int32),
        pltpu.VMEM((16,), jnp.int32),
        pltpu.VMEM((PAD_SLICE,), jnp.int32),
        pltpu.VMEM((16,), jnp.int32),
        pltpu.VMEM((SC_NT * 16,), jnp.int32),
        pltpu.VMEM((NBLK_PAD,), jnp.int32),
        pltpu.VMEM_SHARED((SC_NT * 16,), jnp.int32),
        pltpu.SemaphoreType.DMA,
    ],
)
def _dispatch(eflat_hbm, pos_hbm, stok_hbm, bexp_hbm, *rest):
    _dispatch_body(eflat_hbm, pos_hbm, stok_hbm, bexp_hbm, *rest)


SC_NW = 32
GROWS = PAD_N // SC_NW
GCH = 16
CTOK = T // SC_NW
CCH = 8

_FULL_MESH = plsc.VectorSubcoreMesh(core_axis_name="c", subcore_axis_name="s")


def _gather_body(xt_hbm, stok_hbm, xs_hbm, idx_v, i16_v, rows_v, sem):
    wid = lax.axis_index("s") * 2 + lax.axis_index("c")
    base = wid * GROWS
    pltpu.sync_copy(stok_hbm.at[pl.ds(base, GROWS)], idx_v)
    for j in range(GROWS // GCH):
        i16_v[...] = idx_v[pl.ds(j * GCH, GCH)]
        pltpu.async_copy(xt_hbm.at[i16_v], rows_v, sem).wait()
        pltpu.sync_copy(rows_v, xs_hbm.at[pl.ds(base + j * GCH, GCH)])


@functools.partial(
    pl.kernel,
    out_type=jax.ShapeDtypeStruct((PAD_N, DIM), jnp.float32),
    mesh=_FULL_MESH,
    compiler_params=pltpu.CompilerParams(needs_layout_passes=False),
    scratch_types=[
        pltpu.VMEM((GROWS,), jnp.int32),
        pltpu.VMEM((GCH,), jnp.int32),
        pltpu.VMEM((GCH, DIM), jnp.float32),
        pltpu.SemaphoreType.DMA,
    ],
)
def _sc_gather(xt_hbm, stok_hbm, xs_hbm, *rest):
    _gather_body(xt_hbm, stok_hbm, xs_hbm, *rest)


def _combine_body(ys_hbm, z_hbm, pos_hbm, w_hbm, y_hbm,
                  pos_v, w_v, rows_v, z_v, out_v, sem):
    wid = lax.axis_index("s") * 2 + lax.axis_index("c")
    tbase = wid * CTOK
    pltpu.sync_copy(pos_hbm.at[pl.ds(tbase * TOPK, CTOK * TOPK)], pos_v)
    pltpu.sync_copy(w_hbm.at[pl.ds(tbase * TOPK, CTOK * TOPK)], w_v)
    for j in range(CTOK // CCH):
        p16 = pos_v.at[pl.ds(j * CCH * TOPK, CCH * TOPK)]
        pltpu.async_copy(ys_hbm.at[p16], rows_v, sem).wait()
        pltpu.sync_copy(z_hbm.at[pl.ds(tbase + j * CCH, CCH)], z_v)
        wv = w_v[pl.ds(j * CCH * TOPK, 16)]
        for r in range(CCH):
            w0 = jnp.full((16,), wv[2 * r], jnp.float32)
            w1 = jnp.full((16,), wv[2 * r + 1], jnp.float32)

            def _col(c, carry, r=r, w0=w0, w1=w1):
                a = rows_v[2 * r, pl.ds(c * 16, 16)]
                b = rows_v[2 * r + 1, pl.ds(c * 16, 16)]
                zz = z_v[r, pl.ds(c * 16, 16)]
                out_v[r, pl.ds(c * 16, 16)] = w0 * a + w1 * b + zz
                return carry

            lax.fori_loop(0, DIM // 16, _col, 0)
        pltpu.sync_copy(out_v, y_hbm.at[pl.ds(tbase + j * CCH, CCH)])


@functools.partial(
    pl.kernel,
    out_type=jax.ShapeDtypeStruct((T, DIM), jnp.float32),
    mesh=_FULL_MESH,
    compiler_params=pltpu.CompilerParams(needs_layout_passes=False),
    scratch_types=[
        pltpu.VMEM((CTOK * TOPK,), jnp.int32),
        pltpu.VMEM((CTOK * TOPK,), jnp.float32),
        pltpu.VMEM((CCH * TOPK, DIM), jnp.float32),
        pltpu.VMEM((CCH, DIM), jnp.float32),
        pltpu.VMEM((CCH, DIM), jnp.float32),
        pltpu.SemaphoreType.DMA,
    ],
)
def _sc_combine(ys_hbm, z_hbm, pos_hbm, w_hbm, y_hbm, *rest):
    _combine_body(ys_hbm, z_hbm, pos_hbm, w_hbm, y_hbm, *rest)


def _gate_body(x_ref, gw_ref, gb_ref, idx_ref, w_ref):
    xv = x_ref[...]
    logits = jax.lax.dot_general(
        xv, gw_ref[...], (((1,), (1,)), ((), ())),
        preferred_element_type=jnp.float32)
    m = jnp.max(logits, axis=1, keepdims=True)
    p = jnp.exp(logits - m)
    orig = p / jnp.sum(p, axis=1, keepdims=True)
    s2 = orig + gb_ref[...]
    lane = jax.lax.broadcasted_iota(jnp.int32, (GATE_BT, NEXP), 1)
    m1 = jnp.max(s2, axis=1, keepdims=True)
    idx1 = jnp.min(jnp.where(s2 == m1, lane, NEXP), axis=1, keepdims=True)
    s2m = jnp.where(lane == idx1, -jnp.inf, s2)
    m2 = jnp.max(s2m, axis=1, keepdims=True)
    idx2 = jnp.min(jnp.where(s2m == m2, lane, NEXP), axis=1, keepdims=True)
    w1 = jnp.sum(jnp.where(lane == idx1, orig, 0.0), axis=1, keepdims=True)
    w2 = jnp.sum(jnp.where(lane == idx2, orig, 0.0), axis=1, keepdims=True)
    idx_ref[...] = jnp.concatenate([idx1, idx2], axis=1)
    w_ref[...] = jnp.concatenate([w1, w2], axis=1)


def _gate(xt, gate_w, gate_b):
    return pl.pallas_call(
        _gate_body,
        grid=(T // GATE_BT,),
        in_specs=[
            pl.BlockSpec((GATE_BT, DIM), lambda t: (t, 0)),
            pl.BlockSpec((NEXP, DIM), lambda t: (0, 0)),
            pl.BlockSpec((1, NEXP), lambda t: (0, 0)),
        ],
        out_specs=[
            pl.BlockSpec((GATE_BT, TOPK), lambda t: (t, 0)),
            pl.BlockSpec((GATE_BT, TOPK), lambda t: (t, 0)),
        ],
        out_shape=[
            jax.ShapeDtypeStruct((T, TOPK), jnp.int32),
            jax.ShapeDtypeStruct((T, TOPK), jnp.float32),
        ],
    )(xt, gate_w, gate_b.reshape(1, NEXP))


def _expert_changed(be_ref):
    b = pl.program_id(0)
    return jnp.logical_or(b == 0, be_ref[b] != be_ref[jnp.maximum(b - 1, 0)])


def _gemm_h_body(be_ref, x_ref, w1_ref, w3_ref, h_ref, w1bf, w3bf):
    @pl.when(_expert_changed(be_ref))
    def _():
        w1bf[...] = w1_ref[0].astype(jnp.bfloat16)
        w3bf[...] = w3_ref[0].astype(jnp.bfloat16)

    xv = x_ref[...]
    h1 = jax.lax.dot_general(xv, w1bf[...], (((1,), (1,)), ((), ())),
                             preferred_element_type=jnp.float32)
    h3 = jax.lax.dot_general(xv, w3bf[...], (((1,), (1,)), ((), ())),
                             preferred_element_type=jnp.float32)
    h_ref[...] = (h1 * jax.nn.sigmoid(h1) * h3).astype(jnp.bfloat16)


def _gemm_y_body(be_ref, h_ref, w2_ref, o_ref, w2bf):
    @pl.when(_expert_changed(be_ref))
    def _():
        w2bf[...] = w2_ref[0].astype(jnp.bfloat16)

    o_ref[...] = jax.lax.dot_general(h_ref[...], w2bf[...],
                                     (((1,), (1,)), ((), ())),
                                     preferred_element_type=jnp.float32)


def _grouped_gemm(x_sorted, we1, we3, we2, block_expert):
    h_spec = pltpu.PrefetchScalarGridSpec(
        num_scalar_prefetch=1,
        grid=(NBLK,),
        in_specs=[
            pl.BlockSpec((BROW, DIM), lambda b, be: (b, 0)),
            pl.BlockSpec((1, INTER, DIM), lambda b, be: (be[b], 0, 0)),
            pl.BlockSpec((1, INTER, DIM), lambda b, be: (be[b], 0, 0)),
        ],
        out_specs=pl.BlockSpec((BROW, INTER), lambda b, be: (b, 0)),
        scratch_shapes=[pltpu.VMEM((INTER, DIM), jnp.bfloat16),
                        pltpu.VMEM((INTER, DIM), jnp.bfloat16)],
    )
    h = pl.pallas_call(
        _gemm_h_body,
        grid_spec=h_spec,
        out_shape=jax.ShapeDtypeStruct((PAD_N, INTER), jnp.bfloat16),
    )(block_expert, x_sorted, we1, we3)
    y_spec = pltpu.PrefetchScalarGridSpec(
        num_scalar_prefetch=1,
        grid=(NBLK,),
        in_specs=[
            pl.BlockSpec((BROW, INTER), lambda b, be: (b, 0)),
            pl.BlockSpec((1, DIM, INTER), lambda b, be: (be[b], 0, 0)),
        ],
        out_specs=pl.BlockSpec((BROW, DIM), lambda b, be: (b, 0)),
        scratch_shapes=[pltpu.VMEM((DIM, INTER), jnp.bfloat16)],
    )
    return pl.pallas_call(
        _gemm_y_body,
        grid_spec=y_spec,
        out_shape=jax.ShapeDtypeStruct((PAD_N, DIM), jnp.float32),
    )(block_expert, h, we2)


def _shared_body(x_ref, w1_ref, w3_ref, w2_ref, o_ref, acc_ref,
                 w1bf, w3bf, w2bf):
    i = pl.program_id(0)
    t = pl.program_id(1)

    @pl.when(t == 0)
    def _():
        w1bf[...] = w1_ref[...].astype(jnp.bfloat16)
        w3bf[...] = w3_ref[...].astype(jnp.bfloat16)
        w2bf[...] = w2_ref[...].astype(jnp.bfloat16)

    xv = x_ref[...]
    h1 = jax.lax.dot_general(xv, w1bf[...], (((1,), (1,)), ((), ())),
                             preferred_element_type=jnp.float32)
    h3 = jax.lax.dot_general(xv, w3bf[...], (((1,), (1,)), ((), ())),
                             preferred_element_type=jnp.float32)
    h = (h1 * jax.nn.sigmoid(h1) * h3).astype(jnp.bfloat16)
    part = jax.lax.dot_general(h, w2bf[...], (((1,), (1,)), ((), ())),
                               preferred_element_type=jnp.float32)
    rows = pl.ds(t * SH_BT, SH_BT)

    @pl.when(i == 0)
    def _():
        acc_ref[rows, :] = part

    @pl.when(i > 0)
    def _():
        acc_ref[rows, :] += part

    @pl.when(i == SH_NI - 1)
    def _():
        o_ref[...] = acc_ref[rows, :]


def _shared(xbf, sw1, sw3, sw2):
    return pl.pallas_call(
        _shared_body,
        grid=(SH_NI, T // SH_BT),
        in_specs=[
            pl.BlockSpec((SH_BT, DIM), lambda i, t: (t, 0)),
            pl.BlockSpec((SH_IB, DIM), lambda i, t: (i, 0)),
            pl.BlockSpec((SH_IB, DIM), lambda i, t: (i, 0)),
            pl.BlockSpec((DIM, SH_IB), lambda i, t: (0, i)),
        ],
        out_specs=pl.BlockSpec((SH_BT, DIM), lambda i, t: (t, 0)),
        out_shape=jax.ShapeDtypeStruct((T, DIM), jnp.float32),
        scratch_shapes=[pltpu.VMEM((T, DIM), jnp.float32),
                        pltpu.VMEM((SH_IB, DIM), jnp.bfloat16),
                        pltpu.VMEM((SH_IB, DIM), jnp.bfloat16),
                        pltpu.VMEM((DIM, SH_IB), jnp.bfloat16)],
    )(xbf, sw1, sw3, sw2)


def kernel(x, gate_w, gate_b, we1, we2, we3, sw1, sw2, sw3):
    xt = x.reshape(T, DIM)
    xbf = xt.astype(jnp.bfloat16)
    idx, w = _gate(xt, gate_w, gate_b)

    pos, sorted_token, block_expert = _dispatch(idx.reshape(-1))

    x_sorted = _sc_gather(xt, sorted_token).astype(jnp.bfloat16)
    ys = _grouped_gemm(x_sorted, we1, we3, we2, block_expert)
    z = _shared(xbf, sw1, sw3, sw2)
    y = _sc_combine(ys, z, pos, w.reshape(-1))
    return y.reshape(x.shape)


# speedup vs baseline: 1.1842x; 1.1842x over previous
import functools

import jax
import jax.numpy as jnp
from jax import lax
from jax.experimental import pallas as pl
from jax.experimental.pallas import tpu as pltpu
from jax.experimental.pallas import tpu_sc as plsc

DIM = 2048
INTER = 1408
NEXP = 8
TOPK = 2
SHARED_INTER = 2 * INTER
T = 2048
NASN = T * TOPK
BROW = 256
PAD_N = NASN + NEXP * BROW
NBLK = PAD_N // BROW

GATE_BT = 512
SH_BT = 512
SH_IB = 256
SH_NI = SHARED_INTER // SH_IB


SC_NT = 16
SC_CHUNK = NASN // SC_NT
PAD_SLICE = PAD_N // SC_NT
NBLK_PAD = 32

_DISPATCH_MESH = plsc.VectorSubcoreMesh(
    core_axis_name="c", subcore_axis_name="s", num_cores=1)


def _dispatch_body(eflat_hbm, pos_hbm, stok_hbm, bexp_hbm,
                   e_v, pos_v, tok_v, run_v, ends_v, zero_v, hist_me,
                   hist_all, bev_v, hist_sh, sem):
    wid = lax.axis_index("s")
    lanes = lax.iota(jnp.int32, 16)
    pltpu.sync_copy(eflat_hbm.at[pl.ds(wid * SC_CHUNK, SC_CHUNK)], e_v)

    hist = jnp.zeros((16,), jnp.int32)
    for e in range(NEXP):
        cnt = jnp.zeros((16,), jnp.int32)
        for j in range(SC_CHUNK // 16):
            ev = e_v[pl.ds(j * 16, 16)]
            cnt = cnt + plsc.all_reduce_population_count(ev == e)
        hist = jnp.where(lanes == e, cnt, hist)
    hist_me[...] = hist

    pltpu.sync_copy(hist_me, hist_sh.at[pl.ds(wid * 16, 16)])
    plsc.subcore_barrier()
    pltpu.sync_copy(hist_sh, hist_all)
    counts = jnp.zeros((16,), jnp.int32)
    prefix = jnp.zeros((16,), jnp.int32)
    widv = jnp.full((16,), wid, jnp.int32)
    for t in range(SC_NT):
        row = hist_all[pl.ds(t * 16, 16)]
        counts = counts + row
        prefix = prefix + jnp.where(jnp.full((16,), t, jnp.int32) < widv,
                                    row, 0)
    padded = ((counts + (BROW - 1)) >> 8) << 8
    ends = plsc.cumsum(padded)
    ends_v[...] = ends
    run_v[...] = (ends - padded) + prefix

    ibase = wid * SC_CHUNK
    for j in range(SC_CHUNK // 16):
        ev = e_v[pl.ds(j * 16, 16)]
        blane = plsc.load_gather(run_v, [ev])
        rank = jnp.zeros((16,), jnp.int32)
        newcnt = jnp.zeros((16,), jnp.int32)
        for e in range(NEXP):
            m = ev == e
            cs = plsc.cumsum(m.astype(jnp.int32))
            rank = rank + jnp.where(m, cs - 1, 0)
            newcnt = newcnt + jnp.where(
                lanes == e, plsc.all_reduce_population_count(m), 0)
        pos_v[pl.ds(j * 16, 16)] = blane + rank
        tok_v[pl.ds(j * 16, 16)] = (ibase + j * 16 + lanes) // TOPK
        run_v[...] = run_v[...] + newcnt
    pltpu.sync_copy(pos_v, pos_hbm.at[pl.ds(ibase, SC_CHUNK)])

    for k in range(PAD_SLICE // 16):
        zero_v[pl.ds(k * 16, 16)] = jnp.zeros((16,), jnp.int32)
    pltpu.sync_copy(zero_v, stok_hbm.at[pl.ds(wid * PAD_SLICE, PAD_SLICE)])
    plsc.subcore_barrier()
    pltpu.async_copy(tok_v, stok_hbm.at[pos_v], sem).wait()

    @pl.when(wid == 0)
    def _():
        endsl = ends_v[...]
        for v in range(NBLK_PAD // 16):
            start = (v * 16 + lanes) * BROW
            be = jnp.zeros((16,), jnp.int32)
            for e in range(NEXP):
                be = be + (start >= jnp.full((16,), endsl[e])).astype(
                    jnp.int32)
            bev_v[pl.ds(v * 16, 16)] = jnp.minimum(be, NEXP - 1)
        pltpu.sync_copy(bev_v, bexp_hbm)


@functools.partial(
    pl.kernel,
    out_type=[
        jax.ShapeDtypeStruct((NASN,), jnp.int32),
        jax.ShapeDtypeStruct((PAD_N,), jnp.int32),
        jax.ShapeDtypeStruct((NBLK_PAD,), jnp.int32),
    ],
    mesh=_DISPATCH_MESH,
    compiler_params=pltpu.CompilerParams(needs_layout_passes=False),
    scratch_types=[
        pltpu.VMEM((SC_CHUNK,), jnp.int32),
        pltpu.VMEM((SC_CHUNK,), jnp.int32),
        pltpu.VMEM((SC_CHUNK,), jnp.int32),
        pltpu.VMEM((16,), jnp.int32),
        pltpu.VMEM((16,), jnp.int32),
        pltpu.VMEM((PAD_SLICE,), jnp.int32),
        pltpu.VMEM((16,), jnp.int32),
        pltpu.VMEM((SC_NT * 16,), jnp.int32),
        pltpu.VMEM((NBLK_PAD,), jnp.int32),
        pltpu.VMEM_SHARED((SC_NT * 16,), jnp.int32),
        pltpu.SemaphoreType.DMA,
    ],
)
def _dispatch(eflat_hbm, pos_hbm, stok_hbm, bexp_hbm, *rest):
    _dispatch_body(eflat_hbm, pos_hbm, stok_hbm, bexp_hbm, *rest)


SC_NW = 32
GROWS = PAD_N // SC_NW
GCH = 16
CTOK = T // SC_NW
CCH = 8

_FULL_MESH = plsc.VectorSubcoreMesh(core_axis_name="c", subcore_axis_name="s")


def _gather_body(xt_hbm, stok_hbm, xs_hbm, idx_v, idx2_v, rows_v,
                 g0, g1, g2, w0, w1, w2):
    gs = (g0, g1, g2)
    ws = (w0, w1, w2)
    nch = GROWS // GCH
    wid = lax.axis_index("s") * 2 + lax.axis_index("c")
    base = wid * GROWS
    pltpu.sync_copy(stok_hbm.at[pl.ds(base, GROWS)], idx_v)
    gh = [None] * nch
    wh = [None] * nch
    for j in range(nch):
        b = j % 3
        if j >= 3:
            wh[j - 3].wait()
        idx2_v[b, pl.ds(0, GCH)] = idx_v[pl.ds(j * GCH, GCH)]
        gh[j] = pltpu.async_copy(xt_hbm.at[idx2_v.at[b]], rows_v.at[b], gs[b])
        if j >= 1:
            bp = (j - 1) % 3
            gh[j - 1].wait()
            wh[j - 1] = pltpu.async_copy(
                rows_v.at[bp], xs_hbm.at[pl.ds(base + (j - 1) * GCH, GCH)],
                ws[bp])
    bl = (nch - 1) % 3
    gh[nch - 1].wait()
    wh[nch - 1] = pltpu.async_copy(
        rows_v.at[bl], xs_hbm.at[pl.ds(base + (nch - 1) * GCH, GCH)], ws[bl])
    wh[nch - 3].wait()
    wh[nch - 2].wait()
    wh[nch - 1].wait()


@functools.partial(
    pl.kernel,
    out_type=jax.ShapeDtypeStruct((PAD_N, DIM), jnp.float32),
    mesh=_FULL_MESH,
    compiler_params=pltpu.CompilerParams(needs_layout_passes=False),
    scratch_types=[
        pltpu.VMEM((GROWS,), jnp.int32),
        pltpu.VMEM((3, GCH), jnp.int32),
        pltpu.VMEM((3, GCH, DIM), jnp.float32),
        pltpu.SemaphoreType.DMA,
        pltpu.SemaphoreType.DMA,
        pltpu.SemaphoreType.DMA,
        pltpu.SemaphoreType.DMA,
        pltpu.SemaphoreType.DMA,
        pltpu.SemaphoreType.DMA,
    ],
)
def _sc_gather(xt_hbm, stok_hbm, xs_hbm, *rest):
    _gather_body(xt_hbm, stok_hbm, xs_hbm, *rest)


def _combine_compute(rows_v, z_v, out_v, w_v, b, j):
    wv = w_v[pl.ds(j * CCH * TOPK, 16)]
    for r in range(CCH):
        w0 = jnp.full((16,), wv[2 * r], jnp.float32)
        w1 = jnp.full((16,), wv[2 * r + 1], jnp.float32)

        def _col(c, carry, r=r, w0=w0, w1=w1):
            a = rows_v[b, 2 * r, pl.ds(c * 16, 16)]
            bb = rows_v[b, 2 * r + 1, pl.ds(c * 16, 16)]
            zz = z_v[b, r, pl.ds(c * 16, 16)]
            out_v[r, pl.ds(c * 16, 16)] = w0 * a + w1 * bb + zz
            return carry

        lax.fori_loop(0, DIM // 16, _col, 0)


def _combine_body(ys_hbm, z_hbm, pos_hbm, w_hbm, y_hbm,
                  pos_v, w_v, p2_v, rows_v, z_v, out_v,
                  r0, r1, z0, z1, o0, o1):
    rs = (r0, r1)
    zs = (z0, z1)
    os_ = (o0, o1)
    nch = CTOK // CCH
    wid = lax.axis_index("s") * 2 + lax.axis_index("c")
    tbase = wid * CTOK
    pltpu.sync_copy(pos_hbm.at[pl.ds(tbase * TOPK, CTOK * TOPK)], pos_v)
    pltpu.sync_copy(w_hbm.at[pl.ds(tbase * TOPK, CTOK * TOPK)], w_v)
    rh = [None] * nch
    zh = [None] * nch
    oh = [None] * nch

    def start(j):
        b = j % 2
        p2_v[b, pl.ds(0, CCH * TOPK)] = pos_v[pl.ds(j * CCH * TOPK,
                                                    CCH * TOPK)]
        rh[j] = pltpu.async_copy(ys_hbm.at[p2_v.at[b]], rows_v.at[b], rs[b])
        zh[j] = pltpu.async_copy(z_hbm.at[pl.ds(tbase + j * CCH, CCH)],
                                 z_v.at[b], zs[b])

    def finish(j):
        b = j % 2
        rh[j].wait()
        zh[j].wait()
        if j >= 1:
            oh[j - 1].wait()
        _combine_compute(rows_v, z_v, out_v, w_v, b, j)
        oh[j] = pltpu.async_copy(out_v,
                                 y_hbm.at[pl.ds(tbase + j * CCH, CCH)],
                                 os_[0])

    start(0)
    for j in range(1, nch):
        start(j)
        finish(j - 1)
    finish(nch - 1)
    oh[nch - 1].wait()


@functools.partial(
    pl.kernel,
    out_type=jax.ShapeDtypeStruct((T, DIM), jnp.float32),
    mesh=_FULL_MESH,
    compiler_params=pltpu.CompilerParams(needs_layout_passes=False),
    scratch_types=[
        pltpu.VMEM((CTOK * TOPK,), jnp.int32),
        pltpu.VMEM((CTOK * TOPK,), jnp.float32),
        pltpu.VMEM((2, CCH * TOPK), jnp.int32),
        pltpu.VMEM((2, CCH * TOPK, DIM), jnp.float32),
        pltpu.VMEM((2, CCH, DIM), jnp.float32),
        pltpu.VMEM((CCH, DIM), jnp.float32),
        pltpu.SemaphoreType.DMA,
        pltpu.SemaphoreType.DMA,
        pltpu.SemaphoreType.DMA,
        pltpu.SemaphoreType.DMA,
        pltpu.SemaphoreType.DMA,
        pltpu.SemaphoreType.DMA,
    ],
)
def _sc_combine(ys_hbm, z_hbm, pos_hbm, w_hbm, y_hbm, *rest):
    _combine_body(ys_hbm, z_hbm, pos_hbm, w_hbm, y_hbm, *rest)


def _gate_body(x_ref, gw_ref, gb_ref, idx_ref, w_ref):
    xv = x_ref[...]
    logits = jax.lax.dot_general(
        xv, gw_ref[...], (((1,), (1,)), ((), ())),
        preferred_element_type=jnp.float32)
    m = jnp.max(logits, axis=1, keepdims=True)
    p = jnp.exp(logits - m)
    orig = p / jnp.sum(p, axis=1, keepdims=True)
    s2 = orig + gb_ref[...]
    lane = jax.lax.broadcasted_iota(jnp.int32, (GATE_BT, NEXP), 1)
    m1 = jnp.max(s2, axis=1, keepdims=True)
    idx1 = jnp.min(jnp.where(s2 == m1, lane, NEXP), axis=1, keepdims=True)
    s2m = jnp.where(lane == idx1, -jnp.inf, s2)
    m2 = jnp.max(s2m, axis=1, keepdims=True)
    idx2 = jnp.min(jnp.where(s2m == m2, lane, NEXP), axis=1, keepdims=True)
    w1 = jnp.sum(jnp.where(lane == idx1, orig, 0.0), axis=1, keepdims=True)
    w2 = jnp.sum(jnp.where(lane == idx2, orig, 0.0), axis=1, keepdims=True)
    idx_ref[...] = jnp.concatenate([idx1, idx2], axis=1)
    w_ref[...] = jnp.concatenate([w1, w2], axis=1)


def _gate(xt, gate_w, gate_b):
    return pl.pallas_call(
        _gate_body,
        grid=(T // GATE_BT,),
        in_specs=[
            pl.BlockSpec((GATE_BT, DIM), lambda t: (t, 0)),
            pl.BlockSpec((NEXP, DIM), lambda t: (0, 0)),
            pl.BlockSpec((1, NEXP), lambda t: (0, 0)),
        ],
        out_specs=[
            pl.BlockSpec((GATE_BT, TOPK), lambda t: (t, 0)),
            pl.BlockSpec((GATE_BT, TOPK), lambda t: (t, 0)),
        ],
        out_shape=[
            jax.ShapeDtypeStruct((T, TOPK), jnp.int32),
            jax.ShapeDtypeStruct((T, TOPK), jnp.float32),
        ],
    )(xt, gate_w, gate_b.reshape(1, NEXP))


def _expert_changed(be_ref):
    b = pl.program_id(0)
    return jnp.logical_or(b == 0, be_ref[b] != be_ref[jnp.maximum(b - 1, 0)])


def _gemm_h_body(be_ref, x_ref, w1_ref, w3_ref, h_ref):
    xv = x_ref[...]
    h1 = jax.lax.dot_general(xv, w1_ref[0], (((1,), (1,)), ((), ())),
                             preferred_element_type=jnp.float32)
    h3 = jax.lax.dot_general(xv, w3_ref[0], (((1,), (1,)), ((), ())),
                             preferred_element_type=jnp.float32)
    h_ref[...] = h1 * jax.nn.sigmoid(h1) * h3


def _gemm_y_body(be_ref, h_ref, w2_ref, o_ref):
    o_ref[...] = jax.lax.dot_general(h_ref[...], w2_ref[0],
                                     (((1,), (1,)), ((), ())),
                                     preferred_element_type=jnp.float32)


def _grouped_gemm(x_sorted, we1, we3, we2, block_expert):
    h_spec = pltpu.PrefetchScalarGridSpec(
        num_scalar_prefetch=1,
        grid=(NBLK,),
        in_specs=[
            pl.BlockSpec((BROW, DIM), lambda b, be: (b, 0)),
            pl.BlockSpec((1, INTER, DIM), lambda b, be: (be[b], 0, 0)),
            pl.BlockSpec((1, INTER, DIM), lambda b, be: (be[b], 0, 0)),
        ],
        out_specs=pl.BlockSpec((BROW, INTER), lambda b, be: (b, 0)),
    )
    h = pl.pallas_call(
        _gemm_h_body,
        grid_spec=h_spec,
        out_shape=jax.ShapeDtypeStruct((PAD_N, INTER), jnp.float32),
    )(block_expert, x_sorted, we1, we3)
    y_spec = pltpu.PrefetchScalarGridSpec(
        num_scalar_prefetch=1,
        grid=(NBLK,),
        in_specs=[
            pl.BlockSpec((BROW, INTER), lambda b, be: (b, 0)),
            pl.BlockSpec((1, DIM, INTER), lambda b, be: (be[b], 0, 0)),
        ],
        out_specs=pl.BlockSpec((BROW, DIM), lambda b, be: (b, 0)),
    )
    return pl.pallas_call(
        _gemm_y_body,
        grid_spec=y_spec,
        out_shape=jax.ShapeDtypeStruct((PAD_N, DIM), jnp.float32),
    )(block_expert, h, we2)


def _shared_body(x_ref, w1_ref, w3_ref, w2_ref, o_ref, acc_ref):
    i = pl.program_id(0)
    t = pl.program_id(1)
    xv = x_ref[...]
    h1 = jax.lax.dot_general(xv, w1_ref[...], (((1,), (1,)), ((), ())),
                             preferred_element_type=jnp.float32)
    h3 = jax.lax.dot_general(xv, w3_ref[...], (((1,), (1,)), ((), ())),
                             preferred_element_type=jnp.float32)
    h = h1 * jax.nn.sigmoid(h1) * h3
    part = jax.lax.dot_general(h, w2_ref[...], (((1,), (1,)), ((), ())),
                               preferred_element_type=jnp.float32)
    rows = pl.ds(t * SH_BT, SH_BT)

    @pl.when(i == 0)
    def _():
        acc_ref[rows, :] = part

    @pl.when(i > 0)
    def _():
        acc_ref[rows, :] += part

    @pl.when(i == SH_NI - 1)
    def _():
        o_ref[...] = acc_ref[rows, :]


def _shared(xbf, sw1, sw3, sw2):
    return pl.pallas_call(
        _shared_body,
        grid=(SH_NI, T // SH_BT),
        in_specs=[
            pl.BlockSpec((SH_BT, DIM), lambda i, t: (t, 0)),
            pl.BlockSpec((SH_IB, DIM), lambda i, t: (i, 0)),
            pl.BlockSpec((SH_IB, DIM), lambda i, t: (i, 0)),
            pl.BlockSpec((DIM, SH_IB), lambda i, t: (0, i)),
        ],
        out_specs=pl.BlockSpec((SH_BT, DIM), lambda i, t: (t, 0)),
        out_shape=jax.ShapeDtypeStruct((T, DIM), jnp.float32),
        scratch_shapes=[pltpu.VMEM((T, DIM), jnp.float32)],
    )(xbf, sw1, sw3, sw2)


def kernel(x, gate_w, gate_b, we1, we2, we3, sw1, sw2, sw3):
    xt = x.reshape(T, DIM)
    idx, w = _gate(xt, gate_w, gate_b)

    pos, sorted_token, block_expert = _dispatch(idx.reshape(-1))

    x_sorted = _sc_gather(xt, sorted_token)
    ys = _grouped_gemm(x_sorted, we1, we3, we2, block_expert)
    z = _shared(xt, sw1, sw3, sw2)
    y = _sc_combine(ys, z, pos, w.reshape(-1))
    return y.reshape(x.shape)


# speedup vs baseline: 1.4556x; 1.2292x over previous
import functools

import jax
import jax.numpy as jnp
from jax import lax
from jax.experimental import pallas as pl
from jax.experimental.pallas import tpu as pltpu
from jax.experimental.pallas import tpu_sc as plsc

DIM = 2048
INTER = 1408
NEXP = 8
TOPK = 2
SHARED_INTER = 2 * INTER
T = 2048
NASN = T * TOPK
BROW = 256
PAD_N = NASN + NEXP * BROW
NBLK = PAD_N // BROW

GATE_BT = 512
SH_BT = 512
SH_IB = 256
SH_NI = SHARED_INTER // SH_IB


SC_NT = 16
SC_CHUNK = NASN // SC_NT
PAD_SLICE = PAD_N // SC_NT
NBLK_PAD = 32

_DISPATCH_MESH = plsc.VectorSubcoreMesh(
    core_axis_name="c", subcore_axis_name="s", num_cores=1)


def _dispatch_body(eflat_hbm, pos_hbm, stok_hbm, bexp_hbm,
                   e_v, pos_v, tok_v, run_v, ends_v, zero_v, hist_me,
                   hist_all, bev_v, hist_sh, sem):
    wid = lax.axis_index("s")
    lanes = lax.iota(jnp.int32, 16)
    pltpu.sync_copy(eflat_hbm.at[pl.ds(wid * SC_CHUNK, SC_CHUNK)], e_v)

    hist = jnp.zeros((16,), jnp.int32)
    for e in range(NEXP):
        cnt = jnp.zeros((16,), jnp.int32)
        for j in range(SC_CHUNK // 16):
            ev = e_v[pl.ds(j * 16, 16)]
            cnt = cnt + plsc.all_reduce_population_count(ev == e)
        hist = jnp.where(lanes == e, cnt, hist)
    hist_me[...] = hist

    pltpu.sync_copy(hist_me, hist_sh.at[pl.ds(wid * 16, 16)])
    plsc.subcore_barrier()
    pltpu.sync_copy(hist_sh, hist_all)
    counts = jnp.zeros((16,), jnp.int32)
    prefix = jnp.zeros((16,), jnp.int32)
    widv = jnp.full((16,), wid, jnp.int32)
    for t in range(SC_NT):
        row = hist_all[pl.ds(t * 16, 16)]
        counts = counts + row
        prefix = prefix + jnp.where(jnp.full((16,), t, jnp.int32) < widv,
                                    row, 0)
    padded = ((counts + (BROW - 1)) >> 8) << 8
    ends = plsc.cumsum(padded)
    ends_v[...] = ends
    run_v[...] = (ends - padded) + prefix

    ibase = wid * SC_CHUNK
    for j in range(SC_CHUNK // 16):
        ev = e_v[pl.ds(j * 16, 16)]
        blane = plsc.load_gather(run_v, [ev])
        rank = jnp.zeros((16,), jnp.int32)
        newcnt = jnp.zeros((16,), jnp.int32)
        for e in range(NEXP):
            m = ev == e
            cs = plsc.cumsum(m.astype(jnp.int32))
            rank = rank + jnp.where(m, cs - 1, 0)
            newcnt = newcnt + jnp.where(
                lanes == e, plsc.all_reduce_population_count(m), 0)
        pos_v[pl.ds(j * 16, 16)] = blane + rank
        tok_v[pl.ds(j * 16, 16)] = (ibase + j * 16 + lanes) // TOPK
        run_v[...] = run_v[...] + newcnt
    pltpu.sync_copy(pos_v, pos_hbm.at[pl.ds(ibase, SC_CHUNK)])

    for k in range(PAD_SLICE // 16):
        zero_v[pl.ds(k * 16, 16)] = (
            (wid * PAD_SLICE + k * 16 + lanes) & (T - 1))
    pltpu.sync_copy(zero_v, stok_hbm.at[pl.ds(wid * PAD_SLICE, PAD_SLICE)])
    plsc.subcore_barrier()
    pltpu.async_copy(tok_v, stok_hbm.at[pos_v], sem).wait()

    @pl.when(wid == 0)
    def _():
        endsl = ends_v[...]
        for v in range(NBLK_PAD // 16):
            start = (v * 16 + lanes) * BROW
            be = jnp.zeros((16,), jnp.int32)
            for e in range(NEXP):
                be = be + (start >= jnp.full((16,), endsl[e])).astype(
                    jnp.int32)
            bev_v[pl.ds(v * 16, 16)] = jnp.minimum(be, NEXP - 1)
        pltpu.sync_copy(bev_v, bexp_hbm)


@functools.partial(
    pl.kernel,
    out_type=[
        jax.ShapeDtypeStruct((NASN,), jnp.int32),
        jax.ShapeDtypeStruct((PAD_N,), jnp.int32),
        jax.ShapeDtypeStruct((NBLK_PAD,), jnp.int32),
    ],
    mesh=_DISPATCH_MESH,
    compiler_params=pltpu.CompilerParams(needs_layout_passes=False),
    scratch_types=[
        pltpu.VMEM((SC_CHUNK,), jnp.int32),
        pltpu.VMEM((SC_CHUNK,), jnp.int32),
        pltpu.VMEM((SC_CHUNK,), jnp.int32),
        pltpu.VMEM((16,), jnp.int32),
        pltpu.VMEM((16,), jnp.int32),
        pltpu.VMEM((PAD_SLICE,), jnp.int32),
        pltpu.VMEM((16,), jnp.int32),
        pltpu.VMEM((SC_NT * 16,), jnp.int32),
        pltpu.VMEM((NBLK_PAD,), jnp.int32),
        pltpu.VMEM_SHARED((SC_NT * 16,), jnp.int32),
        pltpu.SemaphoreType.DMA,
    ],
)
def _dispatch(eflat_hbm, pos_hbm, stok_hbm, bexp_hbm, *rest):
    _dispatch_body(eflat_hbm, pos_hbm, stok_hbm, bexp_hbm, *rest)


SC_NW = 32
GROWS = PAD_N // SC_NW
GCH = 16
CTOK = T // SC_NW
CCH = 8

_FULL_MESH = plsc.VectorSubcoreMesh(core_axis_name="c", subcore_axis_name="s")


def _gather_body(xt_hbm, stok_hbm, xs_hbm, idx_v, idx2_v, rows_v,
                 g0, g1, g2, w0, w1, w2):
    gs = (g0, g1, g2)
    ws = (w0, w1, w2)
    nch = GROWS // GCH
    wid = lax.axis_index("s") * 2 + lax.axis_index("c")
    base = wid * GROWS
    pltpu.sync_copy(stok_hbm.at[pl.ds(base, GROWS)], idx_v)
    gh = [None] * nch
    wh = [None] * nch
    for j in range(nch):
        b = j % 3
        if j >= 3:
            wh[j - 3].wait()
        idx2_v[b, pl.ds(0, GCH)] = idx_v[pl.ds(j * GCH, GCH)]
        gh[j] = pltpu.async_copy(xt_hbm.at[idx2_v.at[b]], rows_v.at[b], gs[b])
        if j >= 2:
            bp = (j - 2) % 3
            gh[j - 2].wait()
            wh[j - 2] = pltpu.async_copy(
                rows_v.at[bp], xs_hbm.at[pl.ds(base + (j - 2) * GCH, GCH)],
                ws[bp])
    for j in (nch - 2, nch - 1):
        bp = j % 3
        gh[j].wait()
        wh[j] = pltpu.async_copy(
            rows_v.at[bp], xs_hbm.at[pl.ds(base + j * GCH, GCH)], ws[bp])
    wh[nch - 3].wait()
    wh[nch - 2].wait()
    wh[nch - 1].wait()


@functools.partial(
    pl.kernel,
    out_type=jax.ShapeDtypeStruct((PAD_N, DIM), jnp.float32),
    mesh=_FULL_MESH,
    compiler_params=pltpu.CompilerParams(needs_layout_passes=False),
    scratch_types=[
        pltpu.VMEM((GROWS,), jnp.int32),
        pltpu.VMEM((3, GCH), jnp.int32),
        pltpu.VMEM((3, GCH, DIM), jnp.float32),
        pltpu.SemaphoreType.DMA,
        pltpu.SemaphoreType.DMA,
        pltpu.SemaphoreType.DMA,
        pltpu.SemaphoreType.DMA,
        pltpu.SemaphoreType.DMA,
        pltpu.SemaphoreType.DMA,
    ],
)
def _sc_gather(xt_hbm, stok_hbm, xs_hbm, *rest):
    _gather_body(xt_hbm, stok_hbm, xs_hbm, *rest)


def _combine_compute(rows_v, z_v, out_v, w_v, b, j):
    wv = w_v[pl.ds(j * CCH * TOPK, 16)]
    for r in range(CCH):
        w0 = jnp.full((16,), wv[2 * r], jnp.float32)
        w1 = jnp.full((16,), wv[2 * r + 1], jnp.float32)

        def _col(c, carry, r=r, w0=w0, w1=w1):
            a = rows_v[b, 2 * r, pl.ds(c * 16, 16)]
            bb = rows_v[b, 2 * r + 1, pl.ds(c * 16, 16)]
            zz = z_v[b, r, pl.ds(c * 16, 16)]
            out_v[r, pl.ds(c * 16, 16)] = w0 * a + w1 * bb + zz
            return carry

        lax.fori_loop(0, DIM // 16, _col, 0)


def _combine_body(ys_hbm, z_hbm, pos_hbm, w_hbm, y_hbm,
                  pos_v, w_v, p2_v, rows_v, z_v, out_v,
                  r0, r1, z0, z1, o0, o1):
    rs = (r0, r1)
    zs = (z0, z1)
    os_ = (o0, o1)
    nch = CTOK // CCH
    wid = lax.axis_index("s") * 2 + lax.axis_index("c")
    tbase = wid * CTOK
    pltpu.sync_copy(pos_hbm.at[pl.ds(tbase * TOPK, CTOK * TOPK)], pos_v)
    pltpu.sync_copy(w_hbm.at[pl.ds(tbase * TOPK, CTOK * TOPK)], w_v)
    rh = [None] * nch
    zh = [None] * nch
    oh = [None] * nch

    def start(j):
        b = j % 2
        p2_v[b, pl.ds(0, CCH * TOPK)] = pos_v[pl.ds(j * CCH * TOPK,
                                                    CCH * TOPK)]
        rh[j] = pltpu.async_copy(ys_hbm.at[p2_v.at[b]], rows_v.at[b], rs[b])
        zh[j] = pltpu.async_copy(z_hbm.at[pl.ds(tbase + j * CCH, CCH)],
                                 z_v.at[b], zs[b])

    def finish(j):
        b = j % 2
        rh[j].wait()
        zh[j].wait()
        if j >= 1:
            oh[j - 1].wait()
        _combine_compute(rows_v, z_v, out_v, w_v, b, j)
        oh[j] = pltpu.async_copy(out_v,
                                 y_hbm.at[pl.ds(tbase + j * CCH, CCH)],
                                 os_[0])

    start(0)
    for j in range(1, nch):
        start(j)
        finish(j - 1)
    finish(nch - 1)
    oh[nch - 1].wait()


@functools.partial(
    pl.kernel,
    out_type=jax.ShapeDtypeStruct((T, DIM), jnp.float32),
    mesh=_FULL_MESH,
    compiler_params=pltpu.CompilerParams(needs_layout_passes=False),
    scratch_types=[
        pltpu.VMEM((CTOK * TOPK,), jnp.int32),
        pltpu.VMEM((CTOK * TOPK,), jnp.float32),
        pltpu.VMEM((2, CCH * TOPK), jnp.int32),
        pltpu.VMEM((2, CCH * TOPK, DIM), jnp.float32),
        pltpu.VMEM((2, CCH, DIM), jnp.float32),
        pltpu.VMEM((CCH, DIM), jnp.float32),
        pltpu.SemaphoreType.DMA,
        pltpu.SemaphoreType.DMA,
        pltpu.SemaphoreType.DMA,
        pltpu.SemaphoreType.DMA,
        pltpu.SemaphoreType.DMA,
        pltpu.SemaphoreType.DMA,
    ],
)
def _sc_combine(ys_hbm, z_hbm, pos_hbm, w_hbm, y_hbm, *rest):
    _combine_body(ys_hbm, z_hbm, pos_hbm, w_hbm, y_hbm, *rest)


def _gate_body(x_ref, gw_ref, gb_ref, idx_ref, w_ref):
    xv = x_ref[...]
    logits = jax.lax.dot_general(
        xv, gw_ref[...], (((1,), (1,)), ((), ())),
        preferred_element_type=jnp.float32)
    m = jnp.max(logits, axis=1, keepdims=True)
    p = jnp.exp(logits - m)
    orig = p / jnp.sum(p, axis=1, keepdims=True)
    s2 = orig + gb_ref[...]
    lane = jax.lax.broadcasted_iota(jnp.int32, (GATE_BT, NEXP), 1)
    m1 = jnp.max(s2, axis=1, keepdims=True)
    idx1 = jnp.min(jnp.where(s2 == m1, lane, NEXP), axis=1, keepdims=True)
    s2m = jnp.where(lane == idx1, -jnp.inf, s2)
    m2 = jnp.max(s2m, axis=1, keepdims=True)
    idx2 = jnp.min(jnp.where(s2m == m2, lane, NEXP), axis=1, keepdims=True)
    w1 = jnp.sum(jnp.where(lane == idx1, orig, 0.0), axis=1, keepdims=True)
    w2 = jnp.sum(jnp.where(lane == idx2, orig, 0.0), axis=1, keepdims=True)
    idx_ref[...] = jnp.concatenate([idx1, idx2], axis=1)
    w_ref[...] = jnp.concatenate([w1, w2], axis=1)


def _gate(xt, gate_w, gate_b):
    return pl.pallas_call(
        _gate_body,
        grid=(T // GATE_BT,),
        in_specs=[
            pl.BlockSpec((GATE_BT, DIM), lambda t: (t, 0)),
            pl.BlockSpec((NEXP, DIM), lambda t: (0, 0)),
            pl.BlockSpec((1, NEXP), lambda t: (0, 0)),
        ],
        out_specs=[
            pl.BlockSpec((GATE_BT, TOPK), lambda t: (t, 0)),
            pl.BlockSpec((GATE_BT, TOPK), lambda t: (t, 0)),
        ],
        out_shape=[
            jax.ShapeDtypeStruct((T, TOPK), jnp.int32),
            jax.ShapeDtypeStruct((T, TOPK), jnp.float32),
        ],
    )(xt, gate_w, gate_b.reshape(1, NEXP))


def _expert_changed(be_ref):
    b = pl.program_id(0)
    return jnp.logical_or(b == 0, be_ref[b] != be_ref[jnp.maximum(b - 1, 0)])


def _gemm_h_body(be_ref, x_ref, w1_ref, w3_ref, h_ref):
    xv = x_ref[...]
    h1 = jax.lax.dot_general(xv, w1_ref[0], (((1,), (1,)), ((), ())),
                             preferred_element_type=jnp.float32)
    h3 = jax.lax.dot_general(xv, w3_ref[0], (((1,), (1,)), ((), ())),
                             preferred_element_type=jnp.float32)
    h_ref[...] = h1 * jax.nn.sigmoid(h1) * h3


def _gemm_y_body(be_ref, h_ref, w2_ref, o_ref):
    o_ref[...] = jax.lax.dot_general(h_ref[...], w2_ref[0],
                                     (((1,), (1,)), ((), ())),
                                     preferred_element_type=jnp.float32)


def _grouped_gemm(x_sorted, we1, we3, we2, block_expert):
    h_spec = pltpu.PrefetchScalarGridSpec(
        num_scalar_prefetch=1,
        grid=(NBLK,),
        in_specs=[
            pl.BlockSpec((BROW, DIM), lambda b, be: (b, 0)),
            pl.BlockSpec((1, INTER, DIM), lambda b, be: (be[b], 0, 0)),
            pl.BlockSpec((1, INTER, DIM), lambda b, be: (be[b], 0, 0)),
        ],
        out_specs=pl.BlockSpec((BROW, INTER), lambda b, be: (b, 0)),
    )
    h = pl.pallas_call(
        _gemm_h_body,
        grid_spec=h_spec,
        out_shape=jax.ShapeDtypeStruct((PAD_N, INTER), jnp.float32),
    )(block_expert, x_sorted, we1, we3)
    y_spec = pltpu.PrefetchScalarGridSpec(
        num_scalar_prefetch=1,
        grid=(NBLK,),
        in_specs=[
            pl.BlockSpec((BROW, INTER), lambda b, be: (b, 0)),
            pl.BlockSpec((1, DIM, INTER), lambda b, be: (be[b], 0, 0)),
        ],
        out_specs=pl.BlockSpec((BROW, DIM), lambda b, be: (b, 0)),
    )
    return pl.pallas_call(
        _gemm_y_body,
        grid_spec=y_spec,
        out_shape=jax.ShapeDtypeStruct((PAD_N, DIM), jnp.float32),
    )(block_expert, h, we2)


def _shared_body(x_ref, w1_ref, w3_ref, w2_ref, o_ref, acc_ref):
    i = pl.program_id(0)
    t = pl.program_id(1)
    xv = x_ref[...]
    h1 = jax.lax.dot_general(xv, w1_ref[...], (((1,), (1,)), ((), ())),
                             preferred_element_type=jnp.float32)
    h3 = jax.lax.dot_general(xv, w3_ref[...], (((1,), (1,)), ((), ())),
                             preferred_element_type=jnp.float32)
    h = h1 * jax.nn.sigmoid(h1) * h3
    part = jax.lax.dot_general(h, w2_ref[...], (((1,), (1,)), ((), ())),
                               preferred_element_type=jnp.float32)
    rows = pl.ds(t * SH_BT, SH_BT)

    @pl.when(i == 0)
    def _():
        acc_ref[rows, :] = part

    @pl.when(i > 0)
    def _():
        acc_ref[rows, :] += part

    @pl.when(i == SH_NI - 1)
    def _():
        o_ref[...] = acc_ref[rows, :]


def _shared(xbf, sw1, sw3, sw2):
    return pl.pallas_call(
        _shared_body,
        grid=(SH_NI, T // SH_BT),
        in_specs=[
            pl.BlockSpec((SH_BT, DIM), lambda i, t: (t, 0)),
            pl.BlockSpec((SH_IB, DIM), lambda i, t: (i, 0)),
            pl.BlockSpec((SH_IB, DIM), lambda i, t: (i, 0)),
            pl.BlockSpec((DIM, SH_IB), lambda i, t: (0, i)),
        ],
        out_specs=pl.BlockSpec((SH_BT, DIM), lambda i, t: (t, 0)),
        out_shape=jax.ShapeDtypeStruct((T, DIM), jnp.float32),
        scratch_shapes=[pltpu.VMEM((T, DIM), jnp.float32)],
    )(xbf, sw1, sw3, sw2)


def kernel(x, gate_w, gate_b, we1, we2, we3, sw1, sw2, sw3):
    xt = x.reshape(T, DIM)
    idx, w = _gate(xt, gate_w, gate_b)

    pos, sorted_token, block_expert = _dispatch(idx.reshape(-1))

    x_sorted = _sc_gather(xt, sorted_token)
    ys = _grouped_gemm(x_sorted, we1, we3, we2, block_expert)
    z = _shared(xt, sw1, sw3, sw2)
    y = _sc_combine(ys, z, pos, w.reshape(-1))
    return y.reshape(x.shape)


# speedup vs baseline: 1.5011x; 1.0313x over previous
import functools

import jax
import jax.numpy as jnp
from jax import lax
from jax.experimental import pallas as pl
from jax.experimental.pallas import tpu as pltpu
from jax.experimental.pallas import tpu_sc as plsc

DIM = 2048
INTER = 1408
NEXP = 8
TOPK = 2
SHARED_INTER = 2 * INTER
T = 2048
NASN = T * TOPK
BROW = 256
PAD_N = NASN + NEXP * BROW
NBLK = PAD_N // BROW

GATE_BT = 512
SH_BT = 512
SH_IB = 256
SH_NI = SHARED_INTER // SH_IB


SC_NT = 16
SC_CHUNK = NASN // SC_NT
PAD_SLICE = PAD_N // SC_NT
NBLK_PAD = 32

_DISPATCH_MESH = plsc.VectorSubcoreMesh(
    core_axis_name="c", subcore_axis_name="s", num_cores=1)


def _dispatch_body(eflat_hbm, pos_hbm, stok_hbm, bexp_hbm, bact_hbm,
                   e_v, pos_v, tok_v, run_v, ends_v, rend_v, zero_v, hist_me,
                   hist_all, bev_v, act_v, hist_sh, sem):
    wid = lax.axis_index("s")
    lanes = lax.iota(jnp.int32, 16)
    pltpu.sync_copy(eflat_hbm.at[pl.ds(wid * SC_CHUNK, SC_CHUNK)], e_v)

    hist = jnp.zeros((16,), jnp.int32)
    for e in range(NEXP):
        cnt = jnp.zeros((16,), jnp.int32)
        for j in range(SC_CHUNK // 16):
            ev = e_v[pl.ds(j * 16, 16)]
            cnt = cnt + plsc.all_reduce_population_count(ev == e)
        hist = jnp.where(lanes == e, cnt, hist)
    hist_me[...] = hist

    pltpu.sync_copy(hist_me, hist_sh.at[pl.ds(wid * 16, 16)])
    plsc.subcore_barrier()
    pltpu.sync_copy(hist_sh, hist_all)
    counts = jnp.zeros((16,), jnp.int32)
    prefix = jnp.zeros((16,), jnp.int32)
    widv = jnp.full((16,), wid, jnp.int32)
    for t in range(SC_NT):
        row = hist_all[pl.ds(t * 16, 16)]
        counts = counts + row
        prefix = prefix + jnp.where(jnp.full((16,), t, jnp.int32) < widv,
                                    row, 0)
    padded = ((counts + (BROW - 1)) >> 8) << 8
    ends = plsc.cumsum(padded)
    ends_v[...] = ends
    rend_v[...] = (ends - padded) + counts
    run_v[...] = (ends - padded) + prefix

    ibase = wid * SC_CHUNK
    for j in range(SC_CHUNK // 16):
        ev = e_v[pl.ds(j * 16, 16)]
        blane = plsc.load_gather(run_v, [ev])
        rank = jnp.zeros((16,), jnp.int32)
        newcnt = jnp.zeros((16,), jnp.int32)
        for e in range(NEXP):
            m = ev == e
            cs = plsc.cumsum(m.astype(jnp.int32))
            rank = rank + jnp.where(m, cs - 1, 0)
            newcnt = newcnt + jnp.where(
                lanes == e, plsc.all_reduce_population_count(m), 0)
        pos_v[pl.ds(j * 16, 16)] = blane + rank
        tok_v[pl.ds(j * 16, 16)] = (ibase + j * 16 + lanes) // TOPK
        run_v[...] = run_v[...] + newcnt
    pltpu.sync_copy(pos_v, pos_hbm.at[pl.ds(ibase, SC_CHUNK)])

    for k in range(PAD_SLICE // 16):
        zero_v[pl.ds(k * 16, 16)] = (
            (wid * PAD_SLICE + k * 16 + lanes) & (T - 1))
    pltpu.sync_copy(zero_v, stok_hbm.at[pl.ds(wid * PAD_SLICE, PAD_SLICE)])
    plsc.subcore_barrier()
    pltpu.async_copy(tok_v, stok_hbm.at[pos_v], sem).wait()

    @pl.when(wid == 0)
    def _():
        endsl = ends_v[...]
        for v in range(NBLK_PAD // 16):
            start = (v * 16 + lanes) * BROW
            be = jnp.zeros((16,), jnp.int32)
            for e in range(NEXP):
                be = be + (start >= jnp.full((16,), endsl[e])).astype(
                    jnp.int32)
            be = jnp.minimum(be, NEXP - 1)
            bev_v[pl.ds(v * 16, 16)] = be
            rend = plsc.load_gather(rend_v, [be])
            act_v[pl.ds(v * 16, 16)] = (start < rend).astype(jnp.int32)
        pltpu.sync_copy(bev_v, bexp_hbm)
        pltpu.sync_copy(act_v, bact_hbm)


@functools.partial(
    pl.kernel,
    out_type=[
        jax.ShapeDtypeStruct((NASN,), jnp.int32),
        jax.ShapeDtypeStruct((PAD_N,), jnp.int32),
        jax.ShapeDtypeStruct((NBLK_PAD,), jnp.int32),
        jax.ShapeDtypeStruct((NBLK_PAD,), jnp.int32),
    ],
    mesh=_DISPATCH_MESH,
    compiler_params=pltpu.CompilerParams(needs_layout_passes=False),
    scratch_types=[
        pltpu.VMEM((SC_CHUNK,), jnp.int32),
        pltpu.VMEM((SC_CHUNK,), jnp.int32),
        pltpu.VMEM((SC_CHUNK,), jnp.int32),
        pltpu.VMEM((16,), jnp.int32),
        pltpu.VMEM((16,), jnp.int32),
        pltpu.VMEM((16,), jnp.int32),
        pltpu.VMEM((PAD_SLICE,), jnp.int32),
        pltpu.VMEM((16,), jnp.int32),
        pltpu.VMEM((SC_NT * 16,), jnp.int32),
        pltpu.VMEM((NBLK_PAD,), jnp.int32),
        pltpu.VMEM((NBLK_PAD,), jnp.int32),
        pltpu.VMEM_SHARED((SC_NT * 16,), jnp.int32),
        pltpu.SemaphoreType.DMA,
    ],
)
def _dispatch(eflat_hbm, pos_hbm, stok_hbm, bexp_hbm, bact_hbm, *rest):
    _dispatch_body(eflat_hbm, pos_hbm, stok_hbm, bexp_hbm, bact_hbm, *rest)


SC_NW = 32
GROWS = PAD_N // SC_NW
GCH = 16
CTOK = T // SC_NW
CCH = 8

_FULL_MESH = plsc.VectorSubcoreMesh(core_axis_name="c", subcore_axis_name="s")


def _gather_body(xt_hbm, stok_hbm, xs_hbm, idx_v, idx2_v, rows_v,
                 g0, g1, g2, w0, w1, w2):
    gs = (g0, g1, g2)
    ws = (w0, w1, w2)
    nch = GROWS // GCH
    wid = lax.axis_index("s") * 2 + lax.axis_index("c")
    base = wid * GROWS
    pltpu.sync_copy(stok_hbm.at[pl.ds(base, GROWS)], idx_v)
    gh = [None] * nch
    wh = [None] * nch
    for j in range(nch):
        b = j % 3
        if j >= 3:
            wh[j - 3].wait()
        idx2_v[b, pl.ds(0, GCH)] = idx_v[pl.ds(j * GCH, GCH)]
        gh[j] = pltpu.async_copy(xt_hbm.at[idx2_v.at[b]], rows_v.at[b], gs[b])
        if j >= 2:
            bp = (j - 2) % 3
            gh[j - 2].wait()
            wh[j - 2] = pltpu.async_copy(
                rows_v.at[bp], xs_hbm.at[pl.ds(base + (j - 2) * GCH, GCH)],
                ws[bp])
    for j in (nch - 2, nch - 1):
        bp = j % 3
        gh[j].wait()
        wh[j] = pltpu.async_copy(
            rows_v.at[bp], xs_hbm.at[pl.ds(base + j * GCH, GCH)], ws[bp])
    wh[nch - 3].wait()
    wh[nch - 2].wait()
    wh[nch - 1].wait()


@functools.partial(
    pl.kernel,
    out_type=jax.ShapeDtypeStruct((PAD_N, DIM), jnp.float32),
    mesh=_FULL_MESH,
    compiler_params=pltpu.CompilerParams(needs_layout_passes=False),
    scratch_types=[
        pltpu.VMEM((GROWS,), jnp.int32),
        pltpu.VMEM((3, GCH), jnp.int32),
        pltpu.VMEM((3, GCH, DIM), jnp.float32),
        pltpu.SemaphoreType.DMA,
        pltpu.SemaphoreType.DMA,
        pltpu.SemaphoreType.DMA,
        pltpu.SemaphoreType.DMA,
        pltpu.SemaphoreType.DMA,
        pltpu.SemaphoreType.DMA,
    ],
)
def _sc_gather(xt_hbm, stok_hbm, xs_hbm, *rest):
    _gather_body(xt_hbm, stok_hbm, xs_hbm, *rest)


def _combine_compute(rows_v, z_v, out_v, w_v, b, j):
    wv = w_v[pl.ds(j * CCH * TOPK, 16)]
    for r in range(CCH):
        w0 = jnp.full((16,), wv[2 * r], jnp.float32)
        w1 = jnp.full((16,), wv[2 * r + 1], jnp.float32)

        def _col(c, carry, r=r, w0=w0, w1=w1):
            a = rows_v[b, 2 * r, pl.ds(c * 16, 16)]
            bb = rows_v[b, 2 * r + 1, pl.ds(c * 16, 16)]
            zz = z_v[b, r, pl.ds(c * 16, 16)]
            out_v[r, pl.ds(c * 16, 16)] = w0 * a + w1 * bb + zz
            return carry

        lax.fori_loop(0, DIM // 16, _col, 0)


def _combine_body(ys_hbm, z_hbm, pos_hbm, w_hbm, y_hbm,
                  pos_v, w_v, p2_v, rows_v, z_v, out_v,
                  r0, r1, z0, z1, o0, o1):
    rs = (r0, r1)
    zs = (z0, z1)
    os_ = (o0, o1)
    nch = CTOK // CCH
    wid = lax.axis_index("s") * 2 + lax.axis_index("c")
    tbase = wid * CTOK
    pltpu.sync_copy(pos_hbm.at[pl.ds(tbase * TOPK, CTOK * TOPK)], pos_v)
    pltpu.sync_copy(w_hbm.at[pl.ds(tbase * TOPK, CTOK * TOPK)], w_v)
    rh = [None] * nch
    zh = [None] * nch
    oh = [None] * nch

    def start(j):
        b = j % 2
        p2_v[b, pl.ds(0, CCH * TOPK)] = pos_v[pl.ds(j * CCH * TOPK,
                                                    CCH * TOPK)]
        rh[j] = pltpu.async_copy(ys_hbm.at[p2_v.at[b]], rows_v.at[b], rs[b])
        zh[j] = pltpu.async_copy(z_hbm.at[pl.ds(tbase + j * CCH, CCH)],
                                 z_v.at[b], zs[b])

    def finish(j):
        b = j % 2
        rh[j].wait()
        zh[j].wait()
        if j >= 1:
            oh[j - 1].wait()
        _combine_compute(rows_v, z_v, out_v, w_v, b, j)
        oh[j] = pltpu.async_copy(out_v,
                                 y_hbm.at[pl.ds(tbase + j * CCH, CCH)],
                                 os_[0])

    start(0)
    for j in range(1, nch):
        start(j)
        finish(j - 1)
    finish(nch - 1)
    oh[nch - 1].wait()


@functools.partial(
    pl.kernel,
    out_type=jax.ShapeDtypeStruct((T, DIM), jnp.float32),
    mesh=_FULL_MESH,
    compiler_params=pltpu.CompilerParams(needs_layout_passes=False),
    scratch_types=[
        pltpu.VMEM((CTOK * TOPK,), jnp.int32),
        pltpu.VMEM((CTOK * TOPK,), jnp.float32),
        pltpu.VMEM((2, CCH * TOPK), jnp.int32),
        pltpu.VMEM((2, CCH * TOPK, DIM), jnp.float32),
        pltpu.VMEM((2, CCH, DIM), jnp.float32),
        pltpu.VMEM((CCH, DIM), jnp.float32),
        pltpu.SemaphoreType.DMA,
        pltpu.SemaphoreType.DMA,
        pltpu.SemaphoreType.DMA,
        pltpu.SemaphoreType.DMA,
        pltpu.SemaphoreType.DMA,
        pltpu.SemaphoreType.DMA,
    ],
)
def _sc_combine(ys_hbm, z_hbm, pos_hbm, w_hbm, y_hbm, *rest):
    _combine_body(ys_hbm, z_hbm, pos_hbm, w_hbm, y_hbm, *rest)


def _gate_body(x_ref, gw_ref, gb_ref, idx_ref, w_ref):
    xv = x_ref[...]
    logits = jax.lax.dot_general(
        xv, gw_ref[...], (((1,), (1,)), ((), ())),
        preferred_element_type=jnp.float32)
    m = jnp.max(logits, axis=1, keepdims=True)
    p = jnp.exp(logits - m)
    orig = p / jnp.sum(p, axis=1, keepdims=True)
    s2 = orig + gb_ref[...]
    lane = jax.lax.broadcasted_iota(jnp.int32, (GATE_BT, NEXP), 1)
    m1 = jnp.max(s2, axis=1, keepdims=True)
    idx1 = jnp.min(jnp.where(s2 == m1, lane, NEXP), axis=1, keepdims=True)
    s2m = jnp.where(lane == idx1, -jnp.inf, s2)
    m2 = jnp.max(s2m, axis=1, keepdims=True)
    idx2 = jnp.min(jnp.where(s2m == m2, lane, NEXP), axis=1, keepdims=True)
    w1 = jnp.sum(jnp.where(lane == idx1, orig, 0.0), axis=1, keepdims=True)
    w2 = jnp.sum(jnp.where(lane == idx2, orig, 0.0), axis=1, keepdims=True)
    idx_ref[...] = jnp.concatenate([idx1, idx2], axis=1)
    w_ref[...] = jnp.concatenate([w1, w2], axis=1)


def _gate(xt, gate_w, gate_b):
    return pl.pallas_call(
        _gate_body,
        grid=(T // GATE_BT,),
        in_specs=[
            pl.BlockSpec((GATE_BT, DIM), lambda t: (t, 0)),
            pl.BlockSpec((NEXP, DIM), lambda t: (0, 0)),
            pl.BlockSpec((1, NEXP), lambda t: (0, 0)),
        ],
        out_specs=[
            pl.BlockSpec((GATE_BT, TOPK), lambda t: (t, 0)),
            pl.BlockSpec((GATE_BT, TOPK), lambda t: (t, 0)),
        ],
        out_shape=[
            jax.ShapeDtypeStruct((T, TOPK), jnp.int32),
            jax.ShapeDtypeStruct((T, TOPK), jnp.float32),
        ],
    )(xt, gate_w, gate_b.reshape(1, NEXP))


def _expert_changed(be_ref):
    b = pl.program_id(0)
    return jnp.logical_or(b == 0, be_ref[b] != be_ref[jnp.maximum(b - 1, 0)])


def _gemm_h_body(be_ref, act_ref, x_ref, w1_ref, w3_ref, h_ref):
    @pl.when(act_ref[pl.program_id(0)] == 1)
    def _():
        xv = x_ref[...]
        h1 = jax.lax.dot_general(xv, w1_ref[0], (((1,), (1,)), ((), ())),
                                 preferred_element_type=jnp.float32)
        h3 = jax.lax.dot_general(xv, w3_ref[0], (((1,), (1,)), ((), ())),
                                 preferred_element_type=jnp.float32)
        h_ref[...] = h1 * jax.nn.sigmoid(h1) * h3


def _gemm_y_body(be_ref, act_ref, h_ref, w2_ref, o_ref):
    @pl.when(act_ref[pl.program_id(0)] == 1)
    def _():
        o_ref[...] = jax.lax.dot_general(h_ref[...], w2_ref[0],
                                         (((1,), (1,)), ((), ())),
                                         preferred_element_type=jnp.float32)


def _grouped_gemm(x_sorted, we1, we3, we2, block_expert, block_act):
    h_spec = pltpu.PrefetchScalarGridSpec(
        num_scalar_prefetch=2,
        grid=(NBLK,),
        in_specs=[
            pl.BlockSpec((BROW, DIM), lambda b, be, act: (b, 0)),
            pl.BlockSpec((1, INTER, DIM), lambda b, be, act: (be[b], 0, 0)),
            pl.BlockSpec((1, INTER, DIM), lambda b, be, act: (be[b], 0, 0)),
        ],
        out_specs=pl.BlockSpec((BROW, INTER), lambda b, be, act: (b, 0)),
    )
    h = pl.pallas_call(
        _gemm_h_body,
        grid_spec=h_spec,
        out_shape=jax.ShapeDtypeStruct((PAD_N, INTER), jnp.float32),
    )(block_expert, block_act, x_sorted, we1, we3)
    y_spec = pltpu.PrefetchScalarGridSpec(
        num_scalar_prefetch=2,
        grid=(NBLK,),
        in_specs=[
            pl.BlockSpec((BROW, INTER), lambda b, be, act: (b, 0)),
            pl.BlockSpec((1, DIM, INTER), lambda b, be, act: (be[b], 0, 0)),
        ],
        out_specs=pl.BlockSpec((BROW, DIM), lambda b, be, act: (b, 0)),
    )
    return pl.pallas_call(
        _gemm_y_body,
        grid_spec=y_spec,
        out_shape=jax.ShapeDtypeStruct((PAD_N, DIM), jnp.float32),
    )(block_expert, block_act, h, we2)


def _shared_body(x_ref, w1_ref, w3_ref, w2_ref, o_ref, acc_ref):
    i = pl.program_id(0)
    t = pl.program_id(1)
    xv = x_ref[...]
    h1 = jax.lax.dot_general(xv, w1_ref[...], (((1,), (1,)), ((), ())),
                             preferred_element_type=jnp.float32)
    h3 = jax.lax.dot_general(xv, w3_ref[...], (((1,), (1,)), ((), ())),
                             preferred_element_type=jnp.float32)
    h = h1 * jax.nn.sigmoid(h1) * h3
    part = jax.lax.dot_general(h, w2_ref[...], (((1,), (1,)), ((), ())),
                               preferred_element_type=jnp.float32)
    rows = pl.ds(t * SH_BT, SH_BT)

    @pl.when(i == 0)
    def _():
        acc_ref[rows, :] = part

    @pl.when(i > 0)
    def _():
        acc_ref[rows, :] += part

    @pl.when(i == SH_NI - 1)
    def _():
        o_ref[...] = acc_ref[rows, :]


def _shared(xbf, sw1, sw3, sw2):
    return pl.pallas_call(
        _shared_body,
        grid=(SH_NI, T // SH_BT),
        in_specs=[
            pl.BlockSpec((SH_BT, DIM), lambda i, t: (t, 0)),
            pl.BlockSpec((SH_IB, DIM), lambda i, t: (i, 0)),
            pl.BlockSpec((SH_IB, DIM), lambda i, t: (i, 0)),
            pl.BlockSpec((DIM, SH_IB), lambda i, t: (0, i)),
        ],
        out_specs=pl.BlockSpec((SH_BT, DIM), lambda i, t: (t, 0)),
        out_shape=jax.ShapeDtypeStruct((T, DIM), jnp.float32),
        scratch_shapes=[pltpu.VMEM((T, DIM), jnp.float32)],
    )(xbf, sw1, sw3, sw2)


def kernel(x, gate_w, gate_b, we1, we2, we3, sw1, sw2, sw3):
    xt = x.reshape(T, DIM)
    idx, w = _gate(xt, gate_w, gate_b)

    pos, sorted_token, block_expert, block_act = _dispatch(idx.reshape(-1))

    x_sorted = _sc_gather(xt, sorted_token)
    ys = _grouped_gemm(x_sorted, we1, we3, we2, block_expert, block_act)
    z = _shared(xt, sw1, sw3, sw2)
    y = _sc_combine(ys, z, pos, w.reshape(-1))
    return y.reshape(x.shape)
